# baseline (speedup 1.0000x reference)
import functools

import jax
import jax.numpy as jnp
from jax import lax
from jax.experimental import pallas as pl
from jax.experimental.pallas import tpu as pltpu
from jax.experimental.pallas import tpu_sc as plsc

N = 10000
E = 320000
D = 128
H = 16

NC = 2
NS = 16
NW = NC * NS
CH = 128
NCH = 80
EPW = NCH * CH
EP = NW * EPW
NBUF = 4
NACC = 4
RB = 624
RREM = N - NS * RB


def _mm1_body(x_ref, w_ref, b_ref, o_ref):
    o_ref[...] = lax.dot_general(
        x_ref[...], w_ref[...],
        (((1,), (1,)), ((), ())),
        preferred_element_type=jnp.float32,
    ) + b_ref[...]


def _combine_relu_body(p_ref, o_ref):
    o_ref[...] = jnp.maximum(jnp.sum(p_ref[...], axis=0), 0.0)


def _mm2_body(p_ref, w_ref, b_ref, degp_ref, o_ref):
    s = jnp.sum(p_ref[...], axis=0)
    deg = jnp.sum(degp_ref[...], axis=1, keepdims=True)
    o_ref[...] = lax.dot_general(
        s, w_ref[...],
        (((1,), (1,)), ((), ())),
        preferred_element_type=jnp.float32,
    ) + deg * b_ref[...]


def _make_spmm(compute_deg: bool):
    mesh = plsc.VectorSubcoreMesh(core_axis_name="c", subcore_axis_name="s")

    out_type = [jax.ShapeDtypeStruct((NC * NACC, N, H), jnp.float32)]
    scratch = [
        pltpu.VMEM_SHARED((NACC * N, H), jnp.float32),
    ]
    scratch += [
        pltpu.VMEM((NCH, CH), jnp.int32),
        pltpu.VMEM((NCH, CH), jnp.int32),
        pltpu.VMEM((NCH, CH), jnp.float32),
    ]
    scratch += [pltpu.VMEM((CH, H), jnp.float32) for _ in range(2 * NBUF)]
    scratch += [pltpu.SemaphoreType.DMA for _ in range(2 * NBUF)]
    if compute_deg:
        out_type.append(jax.ShapeDtypeStruct((NC * NACC * N,), jnp.float32))
        scratch.append(pltpu.VMEM_SHARED((NACC * N,), jnp.float32))
        scratch.append(pltpu.VMEM((RB,), jnp.float32))
        scratch.append(pltpu.SemaphoreType.DMA)

    @functools.partial(
        pl.kernel, out_type=out_type, mesh=mesh, scratch_types=scratch,
        compiler_params=pltpu.CompilerParams(use_tc_tiling_on_sc=False))
    def spmm(*refs):
        n_out = 2 if compute_deg else 1
        y_hbm, rows_hbm, cols_hbm, w_hbm, z_hbm = refs[:5]
        out_hbm = refs[5]
        if compute_deg:
            deg_hbm = refs[6]
        k = 5 + n_out
        acc = refs[k]; k += 1
        cols_all, rows_all, w_all = refs[k:k + 3]; k += 3
        g = refs[k:k + NBUF]; k += NBUF
        s = refs[k:k + NBUF]; k += NBUF
        gsem = refs[k:k + NBUF]; k += NBUF
        ssem = refs[k:k + NBUF]; k += NBUF
        if compute_deg:
            deg_v = refs[k]; k += 1
            zb, dsem = refs[k:k + 2]

        cid = lax.axis_index("c")
        sid = lax.axis_index("s")
        wid = sid * NC + cid

        pltpu.sync_copy(cols_hbm.at[wid], cols_all)
        pltpu.sync_copy(rows_hbm.at[wid], rows_all)
        pltpu.sync_copy(w_hbm.at[wid], w_all)

        bank_off = (sid % NACC) * N

        def adj(i, c):
            q, r = i // (CH // 16), i % (CH // 16)
            rows_all[q, pl.ds(r * 16, 16)] = (
                rows_all[q, pl.ds(r * 16, 16)] + bank_off)
            return c

        lax.fori_loop(0, NCH * (CH // 16), adj, 0)

        for a in range(NACC):
            pltpu.sync_copy(z_hbm.at[pl.ds(sid * RB, RB)],
                            acc.at[pl.ds(a * N + sid * RB, RB)])

        @pl.when(sid == 0)
        def _():
            for a in range(NACC):
                pltpu.sync_copy(z_hbm.at[pl.ds(NS * RB, RREM)],
                                acc.at[pl.ds(a * N + NS * RB, RREM)])
        if compute_deg:
            zv = jnp.zeros((16,), jnp.float32)

            def zbody(i, c):
                zb[pl.ds(i * 16, 16)] = zv
                return c

            lax.fori_loop(0, RB // 16, zbody, 0)
            for a in range(NACC):
                pltpu.sync_copy(zb, deg_v.at[pl.ds(a * N + sid * RB, RB)])

            @pl.when(sid == 0)
            def _():
                for a in range(NACC):
                    pltpu.sync_copy(zb.at[pl.ds(0, RREM)],
                                    deg_v.at[pl.ds(a * N + NS * RB, RREM)])

        for b in range(NBUF):
            pltpu.async_copy(y_hbm.at[cols_all.at[b]], g[b], gsem[b])
        plsc.subcore_barrier()

        def process(c, b):
            pltpu.make_async_copy(y_hbm.at[cols_all.at[c]], g[b],
                                  gsem[b]).wait()

            @pl.when(c >= NBUF)
            def _():
                pltpu.make_async_copy(s[b], acc.at[rows_all.at[c - NBUF]],
                                      ssem[b]).wait()

            def sbody(jj, cr):
                wvec = w_all[c, pl.ds(jj * 16, 16)]
                base = jj * 16
                for l in range(16):
                    s[b][base + l, :] = g[b][base + l, :] * wvec[l]
                return cr

            lax.fori_loop(0, CH // 16, sbody, 0)

            @pl.when(c + NBUF < NCH)
            def _():
                pltpu.async_copy(y_hbm.at[cols_all.at[c + NBUF]], g[b],
                                 gsem[b])

            pltpu.async_copy(s[b], acc.at[rows_all.at[c]], ssem[b],
                             add=True)
            if compute_deg:
                pltpu.async_copy(w_all.at[c], deg_v.at[rows_all.at[c]],
                                 dsem, add=True)

        def outer(i, carry):
            for b in range(NBUF):
                process(i * NBUF + b, b)
            return carry

        lax.fori_loop(0, NCH // NBUF, outer, 0)

        for b in range(NBUF):
            pltpu.make_async_copy(s[b], acc.at[rows_all.at[NCH - NBUF + b]],
                                  ssem[b]).wait()
        if compute_deg:
            def dwait(c, carry):
                pltpu.make_async_copy(w_all.at[c], deg_v.at[rows_all.at[c]],
                                      dsem).wait()
                return carry

            lax.fori_loop(0, NCH, dwait, 0)

        plsc.subcore_barrier()
        for a in range(NACC):
            pltpu.sync_copy(acc.at[pl.ds(a * N + sid * RB, RB)],
                            out_hbm.at[cid * NACC + a, pl.ds(sid * RB, RB)])

        @pl.when(sid == 0)
        def _():
            for a in range(NACC):
                pltpu.sync_copy(
                    acc.at[pl.ds(a * N + NS * RB, RREM)],
                    out_hbm.at[cid * NACC + a, pl.ds(NS * RB, RREM)])

        if compute_deg:
            for a in range(NACC):
                off = (cid * NACC + a) * N
                pltpu.sync_copy(deg_v.at[pl.ds(a * N + sid * RB, RB)], zb)
                pltpu.sync_copy(zb, deg_hbm.at[pl.ds(off + sid * RB, RB)])

            @pl.when(sid == 0)
            def _():
                for a in range(NACC):
                    off = (cid * NACC + a) * N
                    pltpu.sync_copy(deg_v.at[pl.ds(a * N + NS * RB, RREM)],
                                    zb.at[pl.ds(0, RREM)])
                    pltpu.sync_copy(zb.at[pl.ds(0, RREM)],
                                    deg_hbm.at[pl.ds(off + NS * RB, RREM)])

    return spmm


_spmm_deg = _make_spmm(True)
_spmm_nodeg = _make_spmm(False)


def kernel(X, edge_index, edge_weight, W1, b1, Wout, bout):
    pad = EP - E
    rows = jnp.concatenate(
        [edge_index[0], jnp.zeros((pad,), jnp.int32)]).reshape(NW, NCH, CH)
    cols = jnp.concatenate(
        [edge_index[1], jnp.zeros((pad,), jnp.int32)]).reshape(NW, NCH, CH)
    w = jnp.concatenate(
        [edge_weight, jnp.zeros((pad,), jnp.float32)]).reshape(NW, NCH, CH)
    zeros = jnp.zeros((N, H), jnp.float32)

    y1 = pl.pallas_call(
        _mm1_body,
        out_shape=jax.ShapeDtypeStruct((N, H), jnp.float32),
    )(X, W1, b1.reshape(1, H))

    p1, deg_parts = _spmm_deg(y1, rows, cols, w, zeros)

    h = pl.pallas_call(
        _combine_relu_body,
        out_shape=jax.ShapeDtypeStruct((N, H), jnp.float32),
    )(p1)

    (p2,) = _spmm_nodeg(h, rows, cols, w, zeros)

    out = pl.pallas_call(
        _mm2_body,
        out_shape=jax.ShapeDtypeStruct((N, D), jnp.float32),
    )(p2, Wout, bout.reshape(1, D), deg_parts.reshape(NC * NACC, N).T)

    return out


# speedup vs baseline: 1.3885x; 1.3885x over previous
import functools

import jax
import jax.numpy as jnp
from jax import lax
from jax.experimental import pallas as pl
from jax.experimental.pallas import tpu as pltpu
from jax.experimental.pallas import tpu_sc as plsc

N = 10000
E = 320000
D = 128
H = 16

NC = 2
NS = 16
NW = NC * NS
CH = 128
NCH = 80
EPW = NCH * CH
EP = NW * EPW
NBUF = 4
NACC = 1
RB = 624
RREM = N - NS * RB


def _mm1_body(x_ref, w_ref, b_ref, o_ref):
    o_ref[...] = lax.dot_general(
        x_ref[...], w_ref[...],
        (((1,), (1,)), ((), ())),
        preferred_element_type=jnp.float32,
    ) + b_ref[...]


def _combine_relu_body(p_ref, o_ref):
    o_ref[...] = jnp.maximum(jnp.sum(p_ref[...], axis=0), 0.0)


def _mm2_body(p_ref, w_ref, b_ref, degp_ref, o_ref):
    s = jnp.sum(p_ref[...], axis=0)
    deg = jnp.sum(degp_ref[...], axis=1, keepdims=True)
    o_ref[...] = lax.dot_general(
        s, w_ref[...],
        (((1,), (1,)), ((), ())),
        preferred_element_type=jnp.float32,
    ) + deg * b_ref[...]


def _make_spmm(compute_deg: bool):
    mesh = plsc.VectorSubcoreMesh(core_axis_name="c", subcore_axis_name="s")

    out_type = [jax.ShapeDtypeStruct((NC * NACC, N, H), jnp.float32)]
    scratch = [
        pltpu.VMEM_SHARED((NACC * N, H), jnp.float32),
    ]
    scratch += [
        pltpu.VMEM((NCH, CH), jnp.int32),
        pltpu.VMEM((NCH, CH), jnp.int32),
        pltpu.VMEM((NCH, CH), jnp.float32),
    ]
    scratch += [pltpu.VMEM((CH, H), jnp.float32) for _ in range(2 * NBUF)]
    scratch += [pltpu.SemaphoreType.DMA for _ in range(2 * NBUF)]
    if compute_deg:
        out_type.append(jax.ShapeDtypeStruct((NC * NACC * N,), jnp.float32))
        scratch.append(pltpu.VMEM_SHARED((NACC * N,), jnp.float32))
        scratch.append(pltpu.VMEM((RB,), jnp.float32))
        scratch.append(pltpu.SemaphoreType.DMA)

    @functools.partial(
        pl.kernel, out_type=out_type, mesh=mesh, scratch_types=scratch,
        compiler_params=pltpu.CompilerParams(use_tc_tiling_on_sc=False))
    def spmm(*refs):
        n_out = 2 if compute_deg else 1
        y_hbm, rows_hbm, cols_hbm, w_hbm, z_hbm = refs[:5]
        out_hbm = refs[5]
        if compute_deg:
            deg_hbm = refs[6]
        k = 5 + n_out
        acc = refs[k]; k += 1
        cols_all, rows_all, w_all = refs[k:k + 3]; k += 3
        g = refs[k:k + NBUF]; k += NBUF
        s = refs[k:k + NBUF]; k += NBUF
        gsem = refs[k:k + NBUF]; k += NBUF
        ssem = refs[k:k + NBUF]; k += NBUF
        if compute_deg:
            deg_v = refs[k]; k += 1
            zb, dsem = refs[k:k + 2]

        cid = lax.axis_index("c")
        sid = lax.axis_index("s")
        wid = sid * NC + cid

        pltpu.sync_copy(cols_hbm.at[wid], cols_all)
        pltpu.sync_copy(rows_hbm.at[wid], rows_all)
        pltpu.sync_copy(w_hbm.at[wid], w_all)

        if NACC > 1:
            bank_off = (sid % NACC) * N

            def adj(i, c):
                q, r = i // (CH // 16), i % (CH // 16)
                rows_all[q, pl.ds(r * 16, 16)] = (
                    rows_all[q, pl.ds(r * 16, 16)] + bank_off)
                return c

            lax.fori_loop(0, NCH * (CH // 16), adj, 0)

        for a in range(NACC):
            pltpu.sync_copy(z_hbm.at[pl.ds(sid * RB, RB)],
                            acc.at[pl.ds(a * N + sid * RB, RB)])

        @pl.when(sid == 0)
        def _():
            for a in range(NACC):
                pltpu.sync_copy(z_hbm.at[pl.ds(NS * RB, RREM)],
                                acc.at[pl.ds(a * N + NS * RB, RREM)])
        if compute_deg:
            zv = jnp.zeros((16,), jnp.float32)

            def zbody(i, c):
                zb[pl.ds(i * 16, 16)] = zv
                return c

            lax.fori_loop(0, RB // 16, zbody, 0)
            for a in range(NACC):
                pltpu.sync_copy(zb, deg_v.at[pl.ds(a * N + sid * RB, RB)])

            @pl.when(sid == 0)
            def _():
                for a in range(NACC):
                    pltpu.sync_copy(zb.at[pl.ds(0, RREM)],
                                    deg_v.at[pl.ds(a * N + NS * RB, RREM)])

        for b in range(NBUF):
            pltpu.async_copy(y_hbm.at[cols_all.at[b]], g[b], gsem[b])
        plsc.subcore_barrier()

        def process(c, b):
            pltpu.make_async_copy(y_hbm.at[cols_all.at[c]], g[b],
                                  gsem[b]).wait()

            @pl.when(c >= NBUF)
            def _():
                pltpu.make_async_copy(s[b], acc.at[rows_all.at[c - NBUF]],
                                      ssem[b]).wait()

            def sbody(jj, cr):
                wvec = w_all[c, pl.ds(jj * 16, 16)]
                base = jj * 16
                for l in range(16):
                    s[b][base + l, :] = g[b][base + l, :] * wvec[l]
                return cr

            lax.fori_loop(0, CH // 16, sbody, 0)

            @pl.when(c + NBUF < NCH)
            def _():
                pltpu.async_copy(y_hbm.at[cols_all.at[c + NBUF]], g[b],
                                 gsem[b])

            pltpu.async_copy(s[b], acc.at[rows_all.at[c]], ssem[b],
                             add=True)
            if compute_deg:
                pltpu.async_copy(w_all.at[c], deg_v.at[rows_all.at[c]],
                                 dsem, add=True)

        def outer(i, carry):
            for b in range(NBUF):
                process(i * NBUF + b, b)
            return carry

        lax.fori_loop(0, NCH // NBUF, outer, 0)

        for b in range(NBUF):
            pltpu.make_async_copy(s[b], acc.at[rows_all.at[NCH - NBUF + b]],
                                  ssem[b]).wait()
        if compute_deg:
            def dwait(c, carry):
                pltpu.make_async_copy(w_all.at[c], deg_v.at[rows_all.at[c]],
                                      dsem).wait()
                return carry

            lax.fori_loop(0, NCH, dwait, 0)

        plsc.subcore_barrier()
        for a in range(NACC):
            pltpu.sync_copy(acc.at[pl.ds(a * N + sid * RB, RB)],
                            out_hbm.at[cid * NACC + a, pl.ds(sid * RB, RB)])

        @pl.when(sid == 0)
        def _():
            for a in range(NACC):
                pltpu.sync_copy(
                    acc.at[pl.ds(a * N + NS * RB, RREM)],
                    out_hbm.at[cid * NACC + a, pl.ds(NS * RB, RREM)])

        if compute_deg:
            for a in range(NACC):
                off = (cid * NACC + a) * N
                pltpu.sync_copy(deg_v.at[pl.ds(a * N + sid * RB, RB)], zb)
                pltpu.sync_copy(zb, deg_hbm.at[pl.ds(off + sid * RB, RB)])

            @pl.when(sid == 0)
            def _():
                for a in range(NACC):
                    off = (cid * NACC + a) * N
                    pltpu.sync_copy(deg_v.at[pl.ds(a * N + NS * RB, RREM)],
                                    zb.at[pl.ds(0, RREM)])
                    pltpu.sync_copy(zb.at[pl.ds(0, RREM)],
                                    deg_hbm.at[pl.ds(off + NS * RB, RREM)])

    return spmm


_spmm_deg = _make_spmm(True)
_spmm_nodeg = _make_spmm(False)


def kernel(X, edge_index, edge_weight, W1, b1, Wout, bout):
    pad = EP - E
    rows = jnp.concatenate(
        [edge_index[0], jnp.zeros((pad,), jnp.int32)]).reshape(NW, NCH, CH)
    cols = jnp.concatenate(
        [edge_index[1], jnp.zeros((pad,), jnp.int32)]).reshape(NW, NCH, CH)
    w = jnp.concatenate(
        [edge_weight, jnp.zeros((pad,), jnp.float32)]).reshape(NW, NCH, CH)
    zeros = jnp.zeros((N, H), jnp.float32)

    y1 = pl.pallas_call(
        _mm1_body,
        out_shape=jax.ShapeDtypeStruct((N, H), jnp.float32),
    )(X, W1, b1.reshape(1, H))

    p1, deg_parts = _spmm_deg(y1, rows, cols, w, zeros)

    h = pl.pallas_call(
        _combine_relu_body,
        out_shape=jax.ShapeDtypeStruct((N, H), jnp.float32),
    )(p1)

    (p2,) = _spmm_nodeg(h, rows, cols, w, zeros)

    out = pl.pallas_call(
        _mm2_body,
        out_shape=jax.ShapeDtypeStruct((N, D), jnp.float32),
    )(p2, Wout, bout.reshape(1, D), deg_parts.reshape(NC * NACC, N).T)

    return out
